# baseline (speedup 1.0000x reference)
import functools

import jax
import jax.numpy as jnp
from jax import lax
from jax.experimental import pallas as pl
from jax.experimental.pallas import tpu as pltpu
from jax.experimental.pallas import tpu_sc as plsc

A = 100000
E_A = 200000
N = 10000
E_C = 160000
ATOM_DIM = 13
MOTIF_DIM = 256
CHAIN_DIM = 256
PROP_DIM = 29
H = 4

_TAIL_BLK = 1000


_SC_CH = 128
_NC, _NS = 2, 16
_NW = _NC * _NS


@functools.lru_cache(maxsize=None)
def _sc_gather_call(Ep, R, D):
    b_per_w = Ep // _NW
    nch = b_per_w // _SC_CH
    mesh = plsc.VectorSubcoreMesh(core_axis_name="c", subcore_axis_name="s")

    @functools.partial(
        pl.kernel, mesh=mesh,
        out_type=jax.ShapeDtypeStruct((Ep, D), jnp.float32),
        scratch_types=[
            pltpu.VMEM((_SC_CH,), jnp.int32),
            pltpu.VMEM((_SC_CH, D), jnp.float32),
            pltpu.SemaphoreType.DMA,
        ],
    )
    def gather_k(table_hbm, idx_hbm, out_hbm, idx_v, rows_v, sem):
        wid = lax.axis_index("s") * _NC + lax.axis_index("c")
        base = wid * b_per_w

        def step(i, carry):
            off = base + i * _SC_CH
            pltpu.sync_copy(idx_hbm.at[pl.ds(off, _SC_CH)], idx_v)
            pltpu.async_copy(table_hbm.at[idx_v], rows_v, sem).wait()
            pltpu.sync_copy(rows_v, out_hbm.at[pl.ds(off, _SC_CH)])
            return carry

        lax.fori_loop(0, nch, step, 0)

    return gather_k


def _sc_gather(table, idx):
    E = idx.shape[0]
    R, D = table.shape
    quantum = _NW * _SC_CH
    Ep = ((E + quantum - 1) // quantum) * quantum
    if Ep != E:
        idx = jnp.pad(idx, (0, Ep - E))
    out = _sc_gather_call(Ep, R, D)(table, idx)
    return out[:E]


def _tail_body(gin_ref, edge_ref, t_ref, vn_ref,
               wv0_ref, bv0_ref, wo0_ref, bo0_ref,
               wv1_ref, bv1_ref, wo1_ref, bo1_ref,
               wv2_ref, bv2_ref, wo2_ref, bo2_ref,
               wgate_ref, bgate_ref, wp1_ref, bp1_ref,
               gamma_ref, beta_ref, wp2_ref, bp2_ref,
               chain_out_ref, prop_out_ref,
               sum_scr, max_scr):
    i = pl.program_id(0)
    nblk = pl.num_programs(0)
    x_gin = gin_ref[...]
    x_edge = edge_ref[...]
    x_t = t_ref[...]
    ch = x_t + vn_ref[...]
    for wv, bv, wo, bo in ((wv0_ref, bv0_ref, wo0_ref, bo0_ref),
                           (wv1_ref, bv1_ref, wo1_ref, bo1_ref),
                           (wv2_ref, bv2_ref, wo2_ref, bo2_ref)):
        hv = jnp.dot(ch, wv[...], preferred_element_type=jnp.float32) + bv[...]
        ch = ch + jnp.dot(hv, wo[...], preferred_element_type=jnp.float32) + bo[...]
    wg = wgate_ref[...]
    glog = (jnp.dot(x_gin, wg[0:CHAIN_DIM], preferred_element_type=jnp.float32)
            + jnp.dot(x_edge, wg[CHAIN_DIM:2 * CHAIN_DIM], preferred_element_type=jnp.float32)
            + jnp.dot(x_t, wg[2 * CHAIN_DIM:3 * CHAIN_DIM], preferred_element_type=jnp.float32)
            + jnp.dot(ch, wg[3 * CHAIN_DIM:], preferred_element_type=jnp.float32)
            + bgate_ref[...])
    lane = jax.lax.broadcasted_iota(jnp.int32, glog.shape, 1)
    glog = jnp.where(lane < 4, glog, -jnp.inf)
    gm = jnp.max(glog, axis=1, keepdims=True)
    ge = jnp.where(lane < 4, jnp.exp(glog - gm), 0.0)
    gw = ge / jnp.sum(ge, axis=1, keepdims=True)
    out = (gw[:, 0:1] * x_gin + gw[:, 1:2] * x_edge
           + gw[:, 2:3] * x_t + gw[:, 3:4] * ch)
    chain_out_ref[...] = out

    psum = jnp.sum(out, axis=0, keepdims=True)
    pmax = jnp.max(out, axis=0, keepdims=True)

    @pl.when(i == 0)
    def _():
        sum_scr[...] = psum
        max_scr[...] = pmax

    @pl.when(i > 0)
    def _():
        sum_scr[...] = sum_scr[...] + psum
        max_scr[...] = jnp.maximum(max_scr[...], pmax)

    @pl.when(i == nblk - 1)
    def _():
        emb = jnp.concatenate([sum_scr[...] * (1.0 / N), max_scr[...]], axis=1)
        h = jnp.dot(emb, wp1_ref[...], preferred_element_type=jnp.float32) + bp1_ref[...]
        h = h * (gamma_ref[...] / jnp.sqrt(1.0 + 1e-5)) + beta_ref[...]
        h = jnp.maximum(h, 0.0)
        pp = jnp.dot(h, wp2_ref[...], preferred_element_type=jnp.float32) + bp2_ref[...]
        prop_out_ref[...] = jnp.broadcast_to(pp, (8, 128))


def _tail(chain_h_gin, chain_h_edge, chain_h_t, virtual_node,
          W_v0, b_v0, W_o0, b_o0, W_v1, b_v1, W_o1, b_o1,
          W_v2, b_v2, W_o2, b_o2, W_gate, b_gate,
          W_p1, b_p1, bn_gamma, bn_beta, W_p2, b_p2):
    nblk = N // _TAIL_BLK
    row = lambda i: (i, 0)
    rep = lambda i: (0, 0)
    bspec = lambda shape: pl.BlockSpec(shape, rep)
    grid_spec = dict(
        grid=(nblk,),
        in_specs=[
            pl.BlockSpec((_TAIL_BLK, CHAIN_DIM), row),
            pl.BlockSpec((_TAIL_BLK, CHAIN_DIM), row),
            pl.BlockSpec((_TAIL_BLK, CHAIN_DIM), row),
            bspec((1, CHAIN_DIM)),
            bspec((CHAIN_DIM, CHAIN_DIM)), bspec((1, CHAIN_DIM)),
            bspec((CHAIN_DIM, CHAIN_DIM)), bspec((1, CHAIN_DIM)),
            bspec((CHAIN_DIM, CHAIN_DIM)), bspec((1, CHAIN_DIM)),
            bspec((CHAIN_DIM, CHAIN_DIM)), bspec((1, CHAIN_DIM)),
            bspec((CHAIN_DIM, CHAIN_DIM)), bspec((1, CHAIN_DIM)),
            bspec((CHAIN_DIM, CHAIN_DIM)), bspec((1, CHAIN_DIM)),
            bspec((4 * CHAIN_DIM, 128)), bspec((1, 128)),
            bspec((2 * CHAIN_DIM, 1024)), bspec((1, 1024)),
            bspec((1, 1024)), bspec((1, 1024)),
            bspec((1024, 128)), bspec((1, 128)),
        ],
        out_specs=[
            pl.BlockSpec((_TAIL_BLK, CHAIN_DIM), row),
            pl.BlockSpec((8, 128), rep),
        ],
    )
    r2 = lambda v: v.reshape(1, -1)
    padl = lambda m, l: jnp.pad(m, ((0, 0), (0, l - m.shape[1])))
    chain_out, prop_pad = pl.pallas_call(
        _tail_body,
        **grid_spec,
        out_shape=[
            jax.ShapeDtypeStruct((N, CHAIN_DIM), jnp.float32),
            jax.ShapeDtypeStruct((8, 128), jnp.float32),
        ],
        scratch_shapes=[
            pltpu.VMEM((1, CHAIN_DIM), jnp.float32),
            pltpu.VMEM((1, CHAIN_DIM), jnp.float32),
        ],
    )(chain_h_gin, chain_h_edge, chain_h_t, virtual_node,
      W_v0, r2(b_v0), W_o0, r2(b_o0), W_v1, r2(b_v1), W_o1, r2(b_o1),
      W_v2, r2(b_v2), W_o2, r2(b_o2), padl(W_gate, 128), padl(r2(b_gate), 128),
      W_p1, r2(b_p1), r2(bn_gamma), r2(bn_beta),
      padl(W_p2, 128), padl(r2(b_p2), 128))
    return chain_out, prop_pad[0:1, 0:PROP_DIM]


def _seg_softmax(logits, seg, num):
    m = jax.ops.segment_max(logits, seg, num_segments=num)
    m = jnp.where(jnp.isfinite(m), m, 0.0)
    e = jnp.exp(logits - m[seg])
    s = jax.ops.segment_sum(e, seg, num_segments=num)
    return e / (s[seg] + 1e-16)


def kernel(atom_feat, atom_edge_index, atom_edge_weight, atom_to_motif,
           chain_edge_index, chain_edge_attr, W_atom, b_atom, W_gat_l,
           W_gat_r, a_gat, b_gat, W_gin, b_gin, W_theta, b_theta, W_phi,
           b_phi, W_q, W_k, W_v, W_e, W_skip, b_skip, virtual_node,
           W_v0, b_v0, W_o0, b_o0, W_v1, b_v1, W_o1, b_o1, W_v2, b_v2,
           W_o2, b_o2, W_gate, b_gate, W_p1, b_p1, bn_gamma, bn_beta,
           W_p2, b_p2):
    asrc = atom_edge_index[0]
    adst = atom_edge_index[1]
    csrc = chain_edge_index[0]
    cdst = chain_edge_index[1]
    seg = atom_to_motif
    w = atom_edge_weight

    hd = MOTIF_DIM // H
    dh = CHAIN_DIM // H
    head_sel = (jax.lax.broadcasted_iota(jnp.int32, (CHAIN_DIM, H), 0) // hd
                == jax.lax.broadcasted_iota(jnp.int32, (CHAIN_DIM, H), 1)
                ).astype(jnp.float32)

    xw = atom_feat @ W_atom
    deg_out = jax.ops.segment_sum(w, asrc, num_segments=A)
    deg_in = jax.ops.segment_sum(w, adst, num_segments=A)
    ns = jax.lax.rsqrt(jnp.clip(deg_out, 1e-6, None))
    nd = jax.lax.rsqrt(jnp.clip(deg_in, 1e-6, None))
    msg = _sc_gather(xw * ns[:, None], asrc) * w[:, None]
    agg = jax.ops.segment_sum(msg, adst, num_segments=A)
    atom_h = jax.nn.relu(agg * nd[:, None] + b_atom)

    hl2 = atom_h @ W_gat_l
    hr2 = atom_h @ W_gat_r
    g_hl_src = _sc_gather(hl2, asrc)
    g_hr_dst = _sc_gather(hr2, adst)
    e2 = jax.nn.leaky_relu(g_hl_src + g_hr_dst, 0.2)
    W_log = a_gat.reshape(-1)[:, None] * head_sel
    logit = jax.lax.dot_general(e2, W_log, (((1,), (0,)), ((), ())),
                                precision=jax.lax.Precision.HIGHEST)
    alpha = _seg_softmax(logit, adst, A)
    mseg = seg[adst]
    motif_num = jax.ops.segment_sum(jnp.repeat(alpha, hd, axis=1) * g_hl_src,
                                    mseg, num_segments=N)
    cnt = jax.ops.segment_sum(jnp.ones((A,), jnp.float32), seg, num_segments=N)
    motif_feats = ((motif_num + cnt[:, None] * b_gat)
                   / jnp.clip(cnt, 1.0, None)[:, None])

    aggc = jax.ops.segment_sum(_sc_gather(motif_feats, csrc), cdst,
                               num_segments=N)
    chain_h_gin = (motif_feats + aggc) @ W_gin + b_gin

    xt = chain_h_gin @ W_theta
    xpd = chain_h_gin @ W_phi - xt + (b_theta + b_phi)
    q2 = chain_h_gin @ W_q
    k2 = chain_h_gin @ W_k
    v2 = chain_h_gin @ W_v
    ee2 = chain_edge_attr @ W_e
    gs = _sc_gather(jnp.concatenate([xt, k2, v2], axis=1), csrc)
    gd = _sc_gather(jnp.concatenate([xpd, q2], axis=1), cdst)
    emsg = gs[:, 0:CHAIN_DIM] + gd[:, 0:CHAIN_DIM]
    chain_h_edge = jax.ops.segment_max(emsg, cdst, num_segments=N)
    chain_h_edge = jnp.where(jnp.isfinite(chain_h_edge), chain_h_edge, 0.0)

    keyv2 = gs[:, CHAIN_DIM:2 * CHAIN_DIM] + ee2
    valv2 = gs[:, 2 * CHAIN_DIM:] + ee2
    tl = jax.lax.dot_general(gd[:, CHAIN_DIM:] * keyv2, head_sel,
                             (((1,), (0,)), ((), ())),
                             precision=jax.lax.Precision.HIGHEST
                             ) / jnp.sqrt(jnp.float32(dh))
    ta = _seg_softmax(tl, cdst, N)
    ta_wide = jnp.repeat(ta, dh, axis=1)
    chain_h_t = (jax.ops.segment_sum(ta_wide * valv2, cdst, num_segments=N)
                 + chain_h_gin @ W_skip + b_skip)

    chain_h, prop_pred = _tail(chain_h_gin, chain_h_edge, chain_h_t,
                               virtual_node,
                               W_v0, b_v0, W_o0, b_o0, W_v1, b_v1, W_o1, b_o1,
                               W_v2, b_v2, W_o2, b_o2, W_gate, b_gate,
                               W_p1, b_p1, bn_gamma, bn_beta, W_p2, b_p2)
    return chain_h, prop_pred


# speedup vs baseline: 1.0423x; 1.0423x over previous
import functools

import jax
import jax.numpy as jnp
from jax import lax
from jax.experimental import pallas as pl
from jax.experimental.pallas import tpu as pltpu
from jax.experimental.pallas import tpu_sc as plsc

A = 100000
E_A = 200000
N = 10000
E_C = 160000
ATOM_DIM = 13
MOTIF_DIM = 256
CHAIN_DIM = 256
PROP_DIM = 29
H = 4

_TAIL_BLK = 1000


_NC, _NS = 2, 16
_NW = _NC * _NS


def _sc_chunk(D):
    return max(8, (45000 // D) // 8 * 8)


@functools.lru_cache(maxsize=None)
def _sc_gather_call(Ep, R, D):
    _SC_CH = _sc_chunk(D)
    b_per_w = Ep // _NW
    nch = b_per_w // _SC_CH
    assert nch >= 2 and nch % 2 == 0
    mesh = plsc.VectorSubcoreMesh(core_axis_name="c", subcore_axis_name="s")

    @functools.partial(
        pl.kernel, mesh=mesh,
        out_type=jax.ShapeDtypeStruct((Ep, D), jnp.float32),
        scratch_types=[
            pltpu.VMEM((b_per_w,), jnp.int32),
            pltpu.VMEM((_SC_CH, D), jnp.float32),
            pltpu.VMEM((_SC_CH, D), jnp.float32),
            pltpu.SemaphoreType.DMA,
            pltpu.SemaphoreType.DMA,
        ],
    )
    def gather_k(table_hbm, idx_hbm, out_hbm, idx_v, rows0_v, rows1_v,
                 sem0, sem1):
        wid = lax.axis_index("s") * _NC + lax.axis_index("c")
        base = wid * b_per_w
        pltpu.sync_copy(idx_hbm.at[pl.ds(base, b_per_w)], idx_v)

        def start(i, rows_v, sem):
            pltpu.make_async_copy(
                table_hbm.at[idx_v.at[pl.ds(i * _SC_CH, _SC_CH)]],
                rows_v, sem).start()

        def fin(i, rows_v, sem):
            pltpu.make_async_copy(
                table_hbm.at[idx_v.at[pl.ds(i * _SC_CH, _SC_CH)]],
                rows_v, sem).wait()
            pltpu.sync_copy(rows_v, out_hbm.at[pl.ds(base + i * _SC_CH,
                                                     _SC_CH)])

        start(0, rows0_v, sem0)

        def pair(j, carry):
            i0 = 2 * j
            start(i0 + 1, rows1_v, sem1)
            fin(i0, rows0_v, sem0)

            @pl.when(i0 + 2 < nch)
            def _():
                start(i0 + 2, rows0_v, sem0)

            fin(i0 + 1, rows1_v, sem1)
            return carry

        lax.fori_loop(0, nch // 2, pair, 0)

    return gather_k


def _sc_gather(table, idx):
    E = idx.shape[0]
    R, D = table.shape
    quantum = _NW * _sc_chunk(D) * 2
    Ep = ((E + quantum - 1) // quantum) * quantum
    if Ep != E:
        idx = jnp.pad(idx, (0, Ep - E))
    out = _sc_gather_call(Ep, R, D)(table, idx)
    return out[:E]


def _tail_body(gin_ref, edge_ref, t_ref, vn_ref,
               wv0_ref, bv0_ref, wo0_ref, bo0_ref,
               wv1_ref, bv1_ref, wo1_ref, bo1_ref,
               wv2_ref, bv2_ref, wo2_ref, bo2_ref,
               wgate_ref, bgate_ref, wp1_ref, bp1_ref,
               gamma_ref, beta_ref, wp2_ref, bp2_ref,
               chain_out_ref, prop_out_ref,
               sum_scr, max_scr):
    i = pl.program_id(0)
    nblk = pl.num_programs(0)
    x_gin = gin_ref[...]
    x_edge = edge_ref[...]
    x_t = t_ref[...]
    ch = x_t + vn_ref[...]
    for wv, bv, wo, bo in ((wv0_ref, bv0_ref, wo0_ref, bo0_ref),
                           (wv1_ref, bv1_ref, wo1_ref, bo1_ref),
                           (wv2_ref, bv2_ref, wo2_ref, bo2_ref)):
        hv = jnp.dot(ch, wv[...], preferred_element_type=jnp.float32) + bv[...]
        ch = ch + jnp.dot(hv, wo[...], preferred_element_type=jnp.float32) + bo[...]
    wg = wgate_ref[...]
    glog = (jnp.dot(x_gin, wg[0:CHAIN_DIM], preferred_element_type=jnp.float32)
            + jnp.dot(x_edge, wg[CHAIN_DIM:2 * CHAIN_DIM], preferred_element_type=jnp.float32)
            + jnp.dot(x_t, wg[2 * CHAIN_DIM:3 * CHAIN_DIM], preferred_element_type=jnp.float32)
            + jnp.dot(ch, wg[3 * CHAIN_DIM:], preferred_element_type=jnp.float32)
            + bgate_ref[...])
    lane = jax.lax.broadcasted_iota(jnp.int32, glog.shape, 1)
    glog = jnp.where(lane < 4, glog, -jnp.inf)
    gm = jnp.max(glog, axis=1, keepdims=True)
    ge = jnp.where(lane < 4, jnp.exp(glog - gm), 0.0)
    gw = ge / jnp.sum(ge, axis=1, keepdims=True)
    out = (gw[:, 0:1] * x_gin + gw[:, 1:2] * x_edge
           + gw[:, 2:3] * x_t + gw[:, 3:4] * ch)
    chain_out_ref[...] = out

    psum = jnp.sum(out, axis=0, keepdims=True)
    pmax = jnp.max(out, axis=0, keepdims=True)

    @pl.when(i == 0)
    def _():
        sum_scr[...] = psum
        max_scr[...] = pmax

    @pl.when(i > 0)
    def _():
        sum_scr[...] = sum_scr[...] + psum
        max_scr[...] = jnp.maximum(max_scr[...], pmax)

    @pl.when(i == nblk - 1)
    def _():
        emb = jnp.concatenate([sum_scr[...] * (1.0 / N), max_scr[...]], axis=1)
        h = jnp.dot(emb, wp1_ref[...], preferred_element_type=jnp.float32) + bp1_ref[...]
        h = h * (gamma_ref[...] / jnp.sqrt(1.0 + 1e-5)) + beta_ref[...]
        h = jnp.maximum(h, 0.0)
        pp = jnp.dot(h, wp2_ref[...], preferred_element_type=jnp.float32) + bp2_ref[...]
        prop_out_ref[...] = jnp.broadcast_to(pp, (8, 128))


def _tail(chain_h_gin, chain_h_edge, chain_h_t, virtual_node,
          W_v0, b_v0, W_o0, b_o0, W_v1, b_v1, W_o1, b_o1,
          W_v2, b_v2, W_o2, b_o2, W_gate, b_gate,
          W_p1, b_p1, bn_gamma, bn_beta, W_p2, b_p2):
    nblk = N // _TAIL_BLK
    row = lambda i: (i, 0)
    rep = lambda i: (0, 0)
    bspec = lambda shape: pl.BlockSpec(shape, rep)
    grid_spec = dict(
        grid=(nblk,),
        in_specs=[
            pl.BlockSpec((_TAIL_BLK, CHAIN_DIM), row),
            pl.BlockSpec((_TAIL_BLK, CHAIN_DIM), row),
            pl.BlockSpec((_TAIL_BLK, CHAIN_DIM), row),
            bspec((1, CHAIN_DIM)),
            bspec((CHAIN_DIM, CHAIN_DIM)), bspec((1, CHAIN_DIM)),
            bspec((CHAIN_DIM, CHAIN_DIM)), bspec((1, CHAIN_DIM)),
            bspec((CHAIN_DIM, CHAIN_DIM)), bspec((1, CHAIN_DIM)),
            bspec((CHAIN_DIM, CHAIN_DIM)), bspec((1, CHAIN_DIM)),
            bspec((CHAIN_DIM, CHAIN_DIM)), bspec((1, CHAIN_DIM)),
            bspec((CHAIN_DIM, CHAIN_DIM)), bspec((1, CHAIN_DIM)),
            bspec((4 * CHAIN_DIM, 128)), bspec((1, 128)),
            bspec((2 * CHAIN_DIM, 1024)), bspec((1, 1024)),
            bspec((1, 1024)), bspec((1, 1024)),
            bspec((1024, 128)), bspec((1, 128)),
        ],
        out_specs=[
            pl.BlockSpec((_TAIL_BLK, CHAIN_DIM), row),
            pl.BlockSpec((8, 128), rep),
        ],
    )
    r2 = lambda v: v.reshape(1, -1)
    padl = lambda m, l: jnp.pad(m, ((0, 0), (0, l - m.shape[1])))
    chain_out, prop_pad = pl.pallas_call(
        _tail_body,
        **grid_spec,
        out_shape=[
            jax.ShapeDtypeStruct((N, CHAIN_DIM), jnp.float32),
            jax.ShapeDtypeStruct((8, 128), jnp.float32),
        ],
        scratch_shapes=[
            pltpu.VMEM((1, CHAIN_DIM), jnp.float32),
            pltpu.VMEM((1, CHAIN_DIM), jnp.float32),
        ],
    )(chain_h_gin, chain_h_edge, chain_h_t, virtual_node,
      W_v0, r2(b_v0), W_o0, r2(b_o0), W_v1, r2(b_v1), W_o1, r2(b_o1),
      W_v2, r2(b_v2), W_o2, r2(b_o2), padl(W_gate, 128), padl(r2(b_gate), 128),
      W_p1, r2(b_p1), r2(bn_gamma), r2(bn_beta),
      padl(W_p2, 128), padl(r2(b_p2), 128))
    return chain_out, prop_pad[0:1, 0:PROP_DIM]


def _seg_softmax(logits, seg, num):
    m = jax.ops.segment_max(logits, seg, num_segments=num)
    m = jnp.where(jnp.isfinite(m), m, 0.0)
    e = jnp.exp(logits - m[seg])
    s = jax.ops.segment_sum(e, seg, num_segments=num)
    return e / (s[seg] + 1e-16)


def kernel(atom_feat, atom_edge_index, atom_edge_weight, atom_to_motif,
           chain_edge_index, chain_edge_attr, W_atom, b_atom, W_gat_l,
           W_gat_r, a_gat, b_gat, W_gin, b_gin, W_theta, b_theta, W_phi,
           b_phi, W_q, W_k, W_v, W_e, W_skip, b_skip, virtual_node,
           W_v0, b_v0, W_o0, b_o0, W_v1, b_v1, W_o1, b_o1, W_v2, b_v2,
           W_o2, b_o2, W_gate, b_gate, W_p1, b_p1, bn_gamma, bn_beta,
           W_p2, b_p2):
    asrc = atom_edge_index[0]
    adst = atom_edge_index[1]
    csrc = chain_edge_index[0]
    cdst = chain_edge_index[1]
    seg = atom_to_motif
    w = atom_edge_weight

    hd = MOTIF_DIM // H
    dh = CHAIN_DIM // H
    head_sel = (jax.lax.broadcasted_iota(jnp.int32, (CHAIN_DIM, H), 0) // hd
                == jax.lax.broadcasted_iota(jnp.int32, (CHAIN_DIM, H), 1)
                ).astype(jnp.float32)

    xw = atom_feat @ W_atom
    deg_out = jax.ops.segment_sum(w, asrc, num_segments=A)
    deg_in = jax.ops.segment_sum(w, adst, num_segments=A)
    ns = jax.lax.rsqrt(jnp.clip(deg_out, 1e-6, None))
    nd = jax.lax.rsqrt(jnp.clip(deg_in, 1e-6, None))
    msg = _sc_gather(xw * ns[:, None], asrc) * w[:, None]
    agg = jax.ops.segment_sum(msg, adst, num_segments=A)
    atom_h = jax.nn.relu(agg * nd[:, None] + b_atom)

    hl2 = atom_h @ W_gat_l
    hr2 = atom_h @ W_gat_r
    g_hl_src = _sc_gather(hl2, asrc)
    g_hr_dst = _sc_gather(hr2, adst)
    e2 = jax.nn.leaky_relu(g_hl_src + g_hr_dst, 0.2)
    W_log = a_gat.reshape(-1)[:, None] * head_sel
    logit = jax.lax.dot_general(e2, W_log, (((1,), (0,)), ((), ())),
                                precision=jax.lax.Precision.HIGHEST)
    alpha = _seg_softmax(logit, adst, A)
    motif_h = (jax.ops.segment_sum(jnp.repeat(alpha, hd, axis=1) * g_hl_src,
                                   adst, num_segments=A) + b_gat)
    cnt = jax.ops.segment_sum(jnp.ones((A,), jnp.float32), seg, num_segments=N)
    motif_feats = (jax.ops.segment_sum(motif_h, seg, num_segments=N)
                   / jnp.clip(cnt, 1.0, None)[:, None])

    aggc = jax.ops.segment_sum(_sc_gather(motif_feats, csrc), cdst,
                               num_segments=N)
    chain_h_gin = (motif_feats + aggc) @ W_gin + b_gin

    xt = chain_h_gin @ W_theta
    xpd = chain_h_gin @ W_phi - xt + (b_theta + b_phi)
    q2 = chain_h_gin @ W_q
    k2 = chain_h_gin @ W_k
    v2 = chain_h_gin @ W_v
    ee2 = chain_edge_attr @ W_e
    gs = _sc_gather(jnp.concatenate([xt, k2, v2], axis=1), csrc)
    gd = _sc_gather(jnp.concatenate([xpd, q2], axis=1), cdst)
    emsg = gs[:, 0:CHAIN_DIM] + gd[:, 0:CHAIN_DIM]
    chain_h_edge = jax.ops.segment_max(emsg, cdst, num_segments=N)
    chain_h_edge = jnp.where(jnp.isfinite(chain_h_edge), chain_h_edge, 0.0)

    keyv2 = gs[:, CHAIN_DIM:2 * CHAIN_DIM] + ee2
    valv2 = gs[:, 2 * CHAIN_DIM:] + ee2
    tl = jax.lax.dot_general(gd[:, CHAIN_DIM:] * keyv2, head_sel,
                             (((1,), (0,)), ((), ())),
                             precision=jax.lax.Precision.HIGHEST
                             ) / jnp.sqrt(jnp.float32(dh))
    ta = _seg_softmax(tl, cdst, N)
    ta_wide = jnp.repeat(ta, dh, axis=1)
    chain_h_t = (jax.ops.segment_sum(ta_wide * valv2, cdst, num_segments=N)
                 + chain_h_gin @ W_skip + b_skip)

    chain_h, prop_pred = _tail(chain_h_gin, chain_h_edge, chain_h_t,
                               virtual_node,
                               W_v0, b_v0, W_o0, b_o0, W_v1, b_v1, W_o1, b_o1,
                               W_v2, b_v2, W_o2, b_o2, W_gate, b_gate,
                               W_p1, b_p1, bn_gamma, bn_beta, W_p2, b_p2)
    return chain_h, prop_pred
